# baseline (speedup 1.0000x reference)
import functools

import jax
import jax.numpy as jnp
from jax import lax
from jax.experimental import pallas as pl
from jax.experimental.pallas import tpu as pltpu
from jax.experimental.pallas import tpu_sc as plsc

NC = 2
NS = 16
L = 16

V = 1000000
VP = 512000
D = 64
B = 4096 * 200
B_PER_W = B // (NC * NS)
C = 128
N_CH = B_PER_W // C
NBUF = 5


def _tc0_body(lo_ref, hi_ref, o_ref):
  eye = jnp.eye(D, dtype=jnp.float32)
  dn = (((0,), (0,)), ((), ()))
  o_ref[:, 0:D] = lax.dot_general(
      lo_ref[...], eye, dn, preferred_element_type=jnp.float32)
  o_ref[:, D:2 * D] = lax.dot_general(
      hi_ref[...], eye, dn, preferred_element_type=jnp.float32)


_tc0 = pl.pallas_call(
    _tc0_body,
    grid=(VP // 1024,),
    in_specs=[
        pl.BlockSpec((D, 1024), lambda i: (0, i)),
        pl.BlockSpec((D, 1024),
                     lambda i: (0, jnp.minimum(VP // 1024 + i, 976))),
    ],
    out_specs=pl.BlockSpec((1024, 2 * D), lambda i: (i, 0)),
    out_shape=jax.ShapeDtypeStruct((VP, 2 * D), jnp.float32),
)


def _tc2_body(i_ref, x_ref, o_ref):
  for si in range(8):
    vt = jnp.transpose(i_ref[si], (1, 0))
    mask = x_ref[pl.ds(si, 1), :] >= VP
    sel = jnp.where(mask, vt[D:2 * D, :], vt[0:D, :])
    o_ref[si] = jnp.maximum(sel, 0.0)


_tc2 = pl.pallas_call(
    _tc2_body,
    grid=(25, 4),
    in_specs=[
        pl.BlockSpec((8, 1024, 2 * D), lambda s, b: (s, b, 0)),
        pl.BlockSpec((8, 1024), lambda s, b: (s, b)),
    ],
    out_specs=pl.BlockSpec((8, D, 1024), lambda s, b: (s, 0, b)),
    out_shape=jax.ShapeDtypeStruct((200, D, 4096), jnp.float32),
)


def _make_sc():
  mesh = plsc.VectorSubcoreMesh(
      core_axis_name="c", subcore_axis_name="s",
      num_cores=NC, num_subcores=NS)

  @functools.partial(
      pl.kernel,
      out_type=jax.ShapeDtypeStruct((B, 2 * D), jnp.float32),
      mesh=mesh,
      compiler_params=pltpu.CompilerParams(use_tc_tiling_on_sc=False),
      scratch_types=[
          pltpu.VMEM((B_PER_W,), jnp.int32),
          pltpu.VMEM((NBUF * C, 2 * D), jnp.float32),
          [pltpu.SemaphoreType.DMA] * NBUF,
          [pltpu.SemaphoreType.DMA] * NBUF,
      ],
  )
  def sc_kernel(w_hbm, idx_hbm, out_hbm, idx_v, buf, gsem, ssem):
    wid = lax.axis_index("s") * NC + lax.axis_index("c")
    base = wid * B_PER_W
    pltpu.sync_copy(idx_hbm.at[pl.ds(base, B_PER_W)], idx_v)

    def gather_desc(c, b):
      return pltpu.make_async_copy(
          w_hbm.at[idx_v.at[pl.ds(c * C, C)]],
          buf.at[pl.ds(b * C, C)],
          gsem[b])

    def scatter_desc(c, b):
      return pltpu.make_async_copy(
          buf.at[pl.ds(b * C, C)],
          out_hbm.at[pl.ds(base + c * C, C)],
          ssem[b])

    gather_desc(0, 0).start()
    gather_desc(1, 1).start()

    @pl.loop(0, N_CH, step=NBUF)
    def _wave(c0):
      for b in range(NBUF):
        c = c0 + b
        gather_desc(c, b).wait()
        scatter_desc(c, b).start()

        @pl.when(c + 2 < N_CH)
        def _():
          b2 = (b + 2) % NBUF

          @pl.when(c + 2 >= NBUF)
          def _():
            scatter_desc(c + 2 - NBUF, b2).wait()

          gather_desc(c + 2, b2).start()

    for c in range(N_CH - NBUF, N_CH):
      scatter_desc(c, c % NBUF).wait()

  return sc_kernel


_sc1 = _make_sc()


@jax.jit
def kernel(x, W):
  wt = W.T
  w2 = _tc0(wt, wt)
  idx = x.astype(jnp.int32).T.reshape(-1)
  pix = jnp.where(idx >= VP, idx - VP, idx)
  im = _sc1(w2, pix)
  im3 = im.reshape(200, 4096, 2 * D)
  xt = x.astype(jnp.int32).T
  out3 = _tc2(im3, xt)
  return out3.transpose(2, 0, 1)


# speedup vs baseline: 1.2611x; 1.2611x over previous
import functools

import jax
import jax.numpy as jnp
from jax import lax
from jax.experimental import pallas as pl
from jax.experimental.pallas import tpu as pltpu
from jax.experimental.pallas import tpu_sc as plsc

NC = 2
NS = 16
L = 16

V = 1000000
VP = 512000
D = 64
B = 4096 * 200
B_PER_W = B // (NC * NS)
C = 128
N_CH = B_PER_W // C
NBUF = 5


def _tc0_body(lo_ref, hi_ref, o_ref):
  o_ref[:, 0:D] = jnp.transpose(lo_ref[...], (1, 0))
  o_ref[:, D:2 * D] = jnp.transpose(hi_ref[...], (1, 0))


_tc0 = pl.pallas_call(
    _tc0_body,
    grid=(VP // 4096,),
    in_specs=[
        pl.BlockSpec((D, 4096), lambda i: (0, i)),
        pl.BlockSpec((D, 4096),
                     lambda i: (0, jnp.minimum(VP // 4096 + i, 244))),
    ],
    out_specs=pl.BlockSpec((4096, 2 * D), lambda i: (i, 0)),
    out_shape=jax.ShapeDtypeStruct((VP, 2 * D), jnp.float32),
)


def _tc2_body(i_ref, x_ref, o_ref):
  for si in range(8):
    vt = jnp.transpose(i_ref[si], (1, 0))
    mask = x_ref[pl.ds(si, 1), :] >= VP
    sel = jnp.where(mask, vt[D:2 * D, :], vt[0:D, :])
    o_ref[si] = jnp.maximum(sel, 0.0)


_tc2 = pl.pallas_call(
    _tc2_body,
    grid=(25, 4),
    in_specs=[
        pl.BlockSpec((8, 1024, 2 * D), lambda s, b: (s, b, 0)),
        pl.BlockSpec((8, 1024), lambda s, b: (s, b)),
    ],
    out_specs=pl.BlockSpec((8, D, 1024), lambda s, b: (s, 0, b)),
    out_shape=jax.ShapeDtypeStruct((200, D, 4096), jnp.float32),
)


def _make_sc():
  mesh = plsc.VectorSubcoreMesh(
      core_axis_name="c", subcore_axis_name="s",
      num_cores=NC, num_subcores=NS)

  @functools.partial(
      pl.kernel,
      out_type=jax.ShapeDtypeStruct((B, 2 * D), jnp.float32),
      mesh=mesh,
      compiler_params=pltpu.CompilerParams(use_tc_tiling_on_sc=False),
      scratch_types=[
          pltpu.VMEM((B_PER_W,), jnp.int32),
          pltpu.VMEM((NBUF * C, 2 * D), jnp.float32),
          [pltpu.SemaphoreType.DMA] * NBUF,
          [pltpu.SemaphoreType.DMA] * NBUF,
      ],
  )
  def sc_kernel(w_hbm, idx_hbm, out_hbm, idx_v, buf, gsem, ssem):
    wid = lax.axis_index("s") * NC + lax.axis_index("c")
    base = wid * B_PER_W
    pltpu.sync_copy(idx_hbm.at[pl.ds(base, B_PER_W)], idx_v)

    def gather_desc(c, b):
      return pltpu.make_async_copy(
          w_hbm.at[idx_v.at[pl.ds(c * C, C)]],
          buf.at[pl.ds(b * C, C)],
          gsem[b])

    def scatter_desc(c, b):
      return pltpu.make_async_copy(
          buf.at[pl.ds(b * C, C)],
          out_hbm.at[pl.ds(base + c * C, C)],
          ssem[b])

    gather_desc(0, 0).start()
    gather_desc(1, 1).start()

    @pl.loop(0, N_CH, step=NBUF)
    def _wave(c0):
      for b in range(NBUF):
        c = c0 + b
        gather_desc(c, b).wait()
        scatter_desc(c, b).start()

        @pl.when(c + 2 < N_CH)
        def _():
          b2 = (b + 2) % NBUF

          @pl.when(c + 2 >= NBUF)
          def _():
            scatter_desc(c + 2 - NBUF, b2).wait()

          gather_desc(c + 2, b2).start()

    for c in range(N_CH - NBUF, N_CH):
      scatter_desc(c, c % NBUF).wait()

  return sc_kernel


_sc1 = _make_sc()


@jax.jit
def kernel(x, W):
  wt = W.T
  w2 = _tc0(wt, wt)
  idx = x.astype(jnp.int32).T.reshape(-1)
  pix = jnp.where(idx >= VP, idx - VP, idx)
  im = _sc1(w2, pix)
  im3 = im.reshape(200, 4096, 2 * D)
  xt = x.astype(jnp.int32).T
  out3 = _tc2(im3, xt)
  return out3.transpose(2, 0, 1)


# speedup vs baseline: 1.3022x; 1.0326x over previous
import functools

import jax
import jax.numpy as jnp
from jax import lax
from jax.experimental import pallas as pl
from jax.experimental.pallas import tpu as pltpu
from jax.experimental.pallas import tpu_sc as plsc

NC = 2
NS = 16
L = 16

V = 1000000
VP = 512000
D = 64
B = 4096 * 200
B_PER_W = B // (NC * NS)
C = 128
N_CH = B_PER_W // C
NBUF = 5


def _tc0_body(lo_ref, hi_ref, o_ref):
  o_ref[:, 0:D] = jnp.transpose(lo_ref[...], (1, 0))
  o_ref[:, D:2 * D] = jnp.transpose(hi_ref[...], (1, 0))


_tc0 = pl.pallas_call(
    _tc0_body,
    grid=(VP // 6400,),
    in_specs=[
        pl.BlockSpec((D, 6400), lambda i: (0, i)),
        pl.BlockSpec((D, 6400),
                     lambda i: (0, jnp.minimum(VP // 6400 + i, 156))),
    ],
    out_specs=pl.BlockSpec((6400, 2 * D), lambda i: (i, 0)),
    out_shape=jax.ShapeDtypeStruct((VP, 2 * D), jnp.float32),
)


def _tc2_body(i_ref, x_ref, o_ref):
  for si in range(8):
    vt = jnp.transpose(i_ref[si], (1, 0))
    mask = x_ref[pl.ds(si, 1), :] >= VP
    sel = jnp.where(mask, vt[D:2 * D, :], vt[0:D, :])
    o_ref[si] = jnp.maximum(sel, 0.0)


_tc2 = pl.pallas_call(
    _tc2_body,
    grid=(25, 4),
    in_specs=[
        pl.BlockSpec((8, 1024, 2 * D), lambda s, b: (s, b, 0)),
        pl.BlockSpec((8, 1024), lambda s, b: (s, b)),
    ],
    out_specs=pl.BlockSpec((8, D, 1024), lambda s, b: (s, 0, b)),
    out_shape=jax.ShapeDtypeStruct((200, D, 4096), jnp.float32),
)


def _make_sc():
  mesh = plsc.VectorSubcoreMesh(
      core_axis_name="c", subcore_axis_name="s",
      num_cores=NC, num_subcores=NS)

  @functools.partial(
      pl.kernel,
      out_type=jax.ShapeDtypeStruct((B, 2 * D), jnp.float32),
      mesh=mesh,
      compiler_params=pltpu.CompilerParams(use_tc_tiling_on_sc=False),
      scratch_types=[
          pltpu.VMEM((B_PER_W,), jnp.int32),
          pltpu.VMEM((NBUF * C, 2 * D), jnp.float32),
          [pltpu.SemaphoreType.DMA] * NBUF,
          [pltpu.SemaphoreType.DMA] * NBUF,
      ],
  )
  def sc_kernel(w_hbm, idx_hbm, out_hbm, idx_v, buf, gsem, ssem):
    wid = lax.axis_index("s") * NC + lax.axis_index("c")
    base = wid * B_PER_W
    pltpu.sync_copy(idx_hbm.at[pl.ds(base, B_PER_W)], idx_v)

    def gather_desc(c, b):
      return pltpu.make_async_copy(
          w_hbm.at[idx_v.at[pl.ds(c * C, C)]],
          buf.at[pl.ds(b * C, C)],
          gsem[b])

    def scatter_desc(c, b):
      return pltpu.make_async_copy(
          buf.at[pl.ds(b * C, C)],
          out_hbm.at[pl.ds(base + c * C, C)],
          ssem[b])

    gather_desc(0, 0).start()
    gather_desc(1, 1).start()

    @pl.loop(0, N_CH, step=NBUF)
    def _wave(c0):
      for b in range(NBUF):
        c = c0 + b
        gather_desc(c, b).wait()
        scatter_desc(c, b).start()

        @pl.when(c + 2 < N_CH)
        def _():
          b2 = (b + 2) % NBUF

          @pl.when(c + 2 >= NBUF)
          def _():
            scatter_desc(c + 2 - NBUF, b2).wait()

          gather_desc(c + 2, b2).start()

    for c in range(N_CH - NBUF, N_CH):
      scatter_desc(c, c % NBUF).wait()

  return sc_kernel


_sc1 = _make_sc()


@jax.jit
def kernel(x, W):
  wt = W.T
  w2 = _tc0(wt, wt)
  idx = x.astype(jnp.int32).T.reshape(-1)
  pix = jnp.where(idx >= VP, idx - VP, idx)
  im = _sc1(w2, pix)
  im3 = im.reshape(200, 4096, 2 * D)
  xt = x.astype(jnp.int32).T
  out3 = _tc2(im3, xt)
  return out3.transpose(2, 0, 1)


# speedup vs baseline: 1.3392x; 1.0284x over previous
import functools

import jax
import jax.numpy as jnp
from jax import lax
from jax.experimental import pallas as pl
from jax.experimental.pallas import tpu as pltpu
from jax.experimental.pallas import tpu_sc as plsc

NC = 2
NS = 16
L = 16

V = 1000000
VP = 512000
D = 64
B = 4096 * 200
B_PER_W = B // (NC * NS)
C = 128
N_CH = B_PER_W // C
NBUF = 5


def _tc0_body(lo_ref, hi_ref, o_ref):
  o_ref[:, 0:D] = jnp.transpose(lo_ref[...], (1, 0))
  o_ref[:, D:2 * D] = jnp.transpose(hi_ref[...], (1, 0))


_tc0 = pl.pallas_call(
    _tc0_body,
    grid=(VP // 12800,),
    in_specs=[
        pl.BlockSpec((D, 12800), lambda i: (0, i)),
        pl.BlockSpec((D, 12800),
                     lambda i: (0, jnp.minimum(VP // 12800 + i, 78))),
    ],
    out_specs=pl.BlockSpec((12800, 2 * D), lambda i: (i, 0)),
    out_shape=jax.ShapeDtypeStruct((VP, 2 * D), jnp.float32),
)


def _tc2_body(i_ref, x_ref, o_ref):
  for si in range(8):
    vt = jnp.transpose(i_ref[si], (1, 0))
    mask = x_ref[pl.ds(si, 1), :] >= VP
    sel = jnp.where(mask, vt[D:2 * D, :], vt[0:D, :])
    o_ref[si] = jnp.maximum(sel, 0.0)


_tc2 = pl.pallas_call(
    _tc2_body,
    grid=(25, 4),
    in_specs=[
        pl.BlockSpec((8, 1024, 2 * D), lambda s, b: (s, b, 0)),
        pl.BlockSpec((8, 1024), lambda s, b: (s, b)),
    ],
    out_specs=pl.BlockSpec((8, D, 1024), lambda s, b: (s, 0, b)),
    out_shape=jax.ShapeDtypeStruct((200, D, 4096), jnp.float32),
)


def _make_sc():
  mesh = plsc.VectorSubcoreMesh(
      core_axis_name="c", subcore_axis_name="s",
      num_cores=NC, num_subcores=NS)

  @functools.partial(
      pl.kernel,
      out_type=jax.ShapeDtypeStruct((B, 2 * D), jnp.float32),
      mesh=mesh,
      compiler_params=pltpu.CompilerParams(use_tc_tiling_on_sc=False),
      scratch_types=[
          pltpu.VMEM((B_PER_W,), jnp.int32),
          pltpu.VMEM((NBUF * C, 2 * D), jnp.float32),
          [pltpu.SemaphoreType.DMA] * NBUF,
          [pltpu.SemaphoreType.DMA] * NBUF,
      ],
  )
  def sc_kernel(w_hbm, idx_hbm, out_hbm, idx_v, buf, gsem, ssem):
    wid = lax.axis_index("s") * NC + lax.axis_index("c")
    base = wid * B_PER_W
    pltpu.sync_copy(idx_hbm.at[pl.ds(base, B_PER_W)], idx_v)

    def gather_desc(c, b):
      return pltpu.make_async_copy(
          w_hbm.at[idx_v.at[pl.ds(c * C, C)]],
          buf.at[pl.ds(b * C, C)],
          gsem[b])

    def scatter_desc(c, b):
      return pltpu.make_async_copy(
          buf.at[pl.ds(b * C, C)],
          out_hbm.at[pl.ds(base + c * C, C)],
          ssem[b])

    gather_desc(0, 0).start()
    gather_desc(1, 1).start()

    @pl.loop(0, N_CH, step=NBUF)
    def _wave(c0):
      for b in range(NBUF):
        c = c0 + b
        gather_desc(c, b).wait()
        scatter_desc(c, b).start()

        @pl.when(c + 2 < N_CH)
        def _():
          b2 = (b + 2) % NBUF

          @pl.when(c + 2 >= NBUF)
          def _():
            scatter_desc(c + 2 - NBUF, b2).wait()

          gather_desc(c + 2, b2).start()

    for c in range(N_CH - NBUF, N_CH):
      scatter_desc(c, c % NBUF).wait()

  return sc_kernel


_sc1 = _make_sc()


@jax.jit
def kernel(x, W):
  wt = W.T
  w2 = _tc0(wt, wt)
  idx = x.astype(jnp.int32).T.reshape(-1)
  pix = jnp.where(idx >= VP, idx - VP, idx)
  im = _sc1(w2, pix)
  im3 = im.reshape(200, 4096, 2 * D)
  xt = x.astype(jnp.int32).T
  out3 = _tc2(im3, xt)
  return out3.transpose(2, 0, 1)
